# baseline (speedup 1.0000x reference)
import functools

import jax
import jax.numpy as jnp
from jax import lax
from jax.experimental import pallas as pl
from jax.experimental.pallas import tpu as pltpu
from jax.experimental.pallas import tpu_sc as plsc

N = 10000
E = 320000
D = 128

NC = 2
NS = 16
NW = NC * NS

C = 128
N_PAD = 10240
E_PAD = 327680
EPT = E_PAD // NW
CHUNKS_MSG = EPT // C
ROWS_PER_SUB = N_PAD // NS

_mesh = plsc.VectorSubcoreMesh(core_axis_name="c", subcore_axis_name="s")

EPS = E_PAD // NS
CHUNKS_DEG = EPS // C


def _sc_degree_body(ei_hbm, out_hbm, idx_v, ones_v, zstage, dacc):
    cid = lax.axis_index("c")
    sid = lax.axis_index("s")

    def fill_ones(i, carry):
        for k in range(8):
            ones_v[i, pl.ds(16 * k, 16)] = jnp.full((16,), 1.0, jnp.float32)
        return carry

    lax.fori_loop(0, C, fill_ones, 0)

    def fill_zero(i, carry):
        for k in range(8):
            zstage[i, pl.ds(16 * k, 16)] = jnp.zeros((16,), jnp.float32)
        return carry

    lax.fori_loop(0, 64, fill_zero, 0)

    def zero_chunk(cch, carry):
        pltpu.sync_copy(zstage,
                        dacc.at[pl.ds(sid * ROWS_PER_SUB + cch * 64, 64)])
        return carry

    lax.fori_loop(0, ROWS_PER_SUB // 64, zero_chunk, 0)
    plsc.subcore_barrier()

    def edge_loop(row):
        def chunk(j, carry):
            off = sid * EPS + j * C
            pltpu.sync_copy(ei_hbm.at[row, pl.ds(off, C)], idx_v)
            pltpu.sync_copy(ones_v, dacc.at[idx_v], add=True)
            return carry

        lax.fori_loop(0, CHUNKS_DEG, chunk, 0)

    @pl.when(cid == 0)
    def _():
        edge_loop(0)

    @pl.when(cid == 1)
    def _():
        edge_loop(1)

    plsc.subcore_barrier()
    pltpu.sync_copy(dacc.at[pl.ds(sid * ROWS_PER_SUB, ROWS_PER_SUB)],
                    out_hbm.at[cid, pl.ds(sid * ROWS_PER_SUB, ROWS_PER_SUB)])


_sc_degree = pl.kernel(
    _sc_degree_body,
    out_type=jax.ShapeDtypeStruct((NC, N_PAD, D), jnp.float32),
    mesh=_mesh,
    scratch_types=[
        pltpu.VMEM((C,), jnp.int32),
        pltpu.VMEM((C, D), jnp.float32),
        pltpu.VMEM((64, D), jnp.float32),
        pltpu.VMEM_SHARED((N_PAD, D), jnp.float32),
    ],
)


def _sc_msgpass_body(h_hbm, ei_hbm, out_hbm, srcs, dsts, rows, zstage, acc,
                     semg0, semg1, *, srow=0, drow=1):
    cid = lax.axis_index("c")
    sid = lax.axis_index("s")
    wid = sid * NC + cid

    def fill_zero(i, carry):
        for k in range(8):
            zstage[i, pl.ds(16 * k, 16)] = jnp.zeros((16,), jnp.float32)
        return carry

    lax.fori_loop(0, 64, fill_zero, 0)

    def zero_chunk(cch, carry):
        pltpu.sync_copy(zstage,
                        acc.at[pl.ds(sid * ROWS_PER_SUB + cch * 64, 64)])
        return carry

    lax.fori_loop(0, ROWS_PER_SUB // 64, zero_chunk, 0)
    plsc.subcore_barrier()

    base = wid * EPT
    sems = (semg0, semg1)

    def load_idx(j, b):
        off = base + j * C
        pltpu.sync_copy(ei_hbm.at[srow, pl.ds(off, C)], srcs.at[b])
        pltpu.sync_copy(ei_hbm.at[drow, pl.ds(off, C)], dsts.at[b])

    def start_gather(b):
        pltpu.async_copy(h_hbm.at[srcs.at[b]], rows.at[b], sems[b])

    def wait_gather(b):
        pltpu.make_async_copy(h_hbm.at[srcs.at[b]], rows.at[b], sems[b]).wait()

    def scatter(b):
        pltpu.sync_copy(rows.at[b], acc.at[dsts.at[b]], add=True)

    load_idx(0, 0)
    start_gather(0)

    def pair(jj, carry):
        j0 = 2 * jj
        load_idx(j0 + 1, 1)
        start_gather(1)
        wait_gather(0)
        scatter(0)

        @pl.when(jj + 1 < CHUNKS_MSG // 2)
        def _():
            load_idx(j0 + 2, 0)
            start_gather(0)

        wait_gather(1)
        scatter(1)
        return carry

    lax.fori_loop(0, CHUNKS_MSG // 2, pair, 0)
    plsc.subcore_barrier()
    pltpu.sync_copy(acc.at[pl.ds(sid * ROWS_PER_SUB, ROWS_PER_SUB)],
                    out_hbm.at[cid, pl.ds(sid * ROWS_PER_SUB, ROWS_PER_SUB)])


def _make_sc_msgpass(srow, drow):
    return pl.kernel(
        functools.partial(_sc_msgpass_body, srow=srow, drow=drow),
        out_type=jax.ShapeDtypeStruct((NC, N_PAD, D), jnp.float32),
        mesh=_mesh,
        scratch_types=[
            pltpu.VMEM((2, C), jnp.int32),
            pltpu.VMEM((2, C), jnp.int32),
            pltpu.VMEM((2, C, D), jnp.float32),
            pltpu.VMEM((64, D), jnp.float32),
            pltpu.VMEM_SHARED((N_PAD, D), jnp.float32),
            pltpu.SemaphoreType.DMA,
            pltpu.SemaphoreType.DMA,
        ],
    )


_sc_msgpass = _make_sc_msgpass(0, 1)

_BLK = 1024
_GRID = N_PAD // _BLK


def _tc_prescale_body(x_ref, deg_ref, o_ref):
    o_ref[...] = x_ref[...] * lax.rsqrt(jnp.maximum(deg_ref[0], 1.0))


_tc_prescale = pl.pallas_call(
    _tc_prescale_body,
    grid=(_GRID,),
    in_specs=[
        pl.BlockSpec((_BLK, D), lambda i: (i, 0)),
        pl.BlockSpec((NC, _BLK, D), lambda i: (0, i, 0)),
    ],
    out_specs=pl.BlockSpec((_BLK, D), lambda i: (i, 0)),
    out_shape=jax.ShapeDtypeStruct((N_PAD, D), jnp.float32),
)


def _tc_layer_body(p_ref, deg_ref, w_ref, b_ref, o_ref, *, last):
    m = (p_ref[0] + p_ref[1]) * lax.rsqrt(jnp.maximum(deg_ref[1], 1.0))
    y = jnp.dot(m, w_ref[...], preferred_element_type=jnp.float32) + b_ref[...]
    if last:
        o_ref[...] = jnp.max(y, axis=1, keepdims=True)
    else:
        o_ref[...] = jnp.maximum(y, 0.0) * lax.rsqrt(
            jnp.maximum(deg_ref[0], 1.0))


def _make_tc_layer(last):
    return pl.pallas_call(
        functools.partial(_tc_layer_body, last=last),
        grid=(_GRID,),
        in_specs=[
            pl.BlockSpec((NC, _BLK, D), lambda i: (0, i, 0)),
            pl.BlockSpec((NC, _BLK, D), lambda i: (0, i, 0)),
            pl.BlockSpec((D, D), lambda i: (0, 0)),
            pl.BlockSpec((1, D), lambda i: (0, 0)),
        ],
        out_specs=pl.BlockSpec((_BLK, 1 if last else D), lambda i: (i, 0)),
        out_shape=jax.ShapeDtypeStruct((N_PAD, 1 if last else D), jnp.float32),
    )


_tc_layer_mid = _make_tc_layer(last=False)
_tc_layer_last = _make_tc_layer(last=True)


def kernel(x, edge_index, W1, b1, W2, b2, W3, b3, W4, b4):
    ei = jnp.concatenate(
        [edge_index, jnp.full((2, E_PAD - E), N, dtype=jnp.int32)], axis=1)
    x_pad = jnp.concatenate(
        [x, jnp.zeros((N_PAD - N, D), dtype=jnp.float32)], axis=0)
    deg = _sc_degree(ei)
    h = _tc_prescale(x_pad, deg)
    for w, b, last in ((W1, b1, False), (W2, b2, False), (W3, b3, False),
                       (W4, b4, True)):
        p = _sc_msgpass(h, ei)
        layer = _tc_layer_last if last else _tc_layer_mid
        h = layer(p, deg, w, b.reshape(1, D))
    return h[:N, 0]


# speedup vs baseline: 1.0266x; 1.0266x over previous
import functools

import jax
import jax.numpy as jnp
from jax import lax
from jax.experimental import pallas as pl
from jax.experimental.pallas import tpu as pltpu
from jax.experimental.pallas import tpu_sc as plsc

N = 10000
E = 320000
D = 128

NC = 2
NS = 16
NW = NC * NS

C = 128
N_PAD = 10240
E_PAD = 327680
EPT = E_PAD // NW
CHUNKS_MSG = EPT // C
ROWS_PER_SUB = N_PAD // NS

_mesh = plsc.VectorSubcoreMesh(core_axis_name="c", subcore_axis_name="s")

EPS = E_PAD // NS
CHUNKS_DEG = EPS // C


def _sc_degree_body(ei_hbm, out_hbm, idx_v, ones_v, zstage, dacc):
    cid = lax.axis_index("c")
    sid = lax.axis_index("s")

    def fill_ones(i, carry):
        for k in range(8):
            ones_v[i, pl.ds(16 * k, 16)] = jnp.full((16,), 1.0, jnp.float32)
        return carry

    lax.fori_loop(0, C, fill_ones, 0)

    def fill_zero(i, carry):
        for k in range(8):
            zstage[i, pl.ds(16 * k, 16)] = jnp.zeros((16,), jnp.float32)
        return carry

    lax.fori_loop(0, 64, fill_zero, 0)

    def zero_chunk(cch, carry):
        pltpu.sync_copy(zstage,
                        dacc.at[pl.ds(sid * ROWS_PER_SUB + cch * 64, 64)])
        return carry

    lax.fori_loop(0, ROWS_PER_SUB // 64, zero_chunk, 0)
    plsc.subcore_barrier()

    def edge_loop(row):
        def chunk(j, carry):
            pltpu.sync_copy(ei_hbm.at[row, sid * CHUNKS_DEG + j], idx_v)
            pltpu.sync_copy(ones_v, dacc.at[idx_v], add=True)
            return carry

        lax.fori_loop(0, CHUNKS_DEG, chunk, 0)

    @pl.when(cid == 0)
    def _():
        edge_loop(0)

    @pl.when(cid == 1)
    def _():
        edge_loop(1)

    plsc.subcore_barrier()
    pltpu.sync_copy(dacc.at[pl.ds(sid * ROWS_PER_SUB, ROWS_PER_SUB)],
                    out_hbm.at[cid, pl.ds(sid * ROWS_PER_SUB, ROWS_PER_SUB)])


_sc_degree = pl.kernel(
    _sc_degree_body,
    out_type=jax.ShapeDtypeStruct((NC, N_PAD, D), jnp.float32),
    mesh=_mesh,
    scratch_types=[
        pltpu.VMEM((C,), jnp.int32),
        pltpu.VMEM((C, D), jnp.float32),
        pltpu.VMEM((64, D), jnp.float32),
        pltpu.VMEM_SHARED((N_PAD, D), jnp.float32),
    ],
)


def _sc_msgpass_body(h_hbm, ei_hbm, zeros_hbm, out_hbm, srcs, dsts_pre, rows,
                     acc, semg0, semg1, *, srow=0, drow=1):
    cid = lax.axis_index("c")
    sid = lax.axis_index("s")
    wid = sid * NC + cid
    pltpu.sync_copy(zeros_hbm.at[pl.ds(sid * ROWS_PER_SUB, ROWS_PER_SUB)],
                    acc.at[pl.ds(sid * ROWS_PER_SUB, ROWS_PER_SUB)])
    cb = wid * CHUNKS_MSG
    pltpu.sync_copy(ei_hbm.at[drow, pl.ds(cb, CHUNKS_MSG)], dsts_pre)
    plsc.subcore_barrier()

    sems = (semg0, semg1)

    def load_src(j, b):
        pltpu.sync_copy(ei_hbm.at[srow, cb + j], srcs.at[b])

    def start_gather(b):
        pltpu.async_copy(h_hbm.at[srcs.at[b]], rows.at[b], sems[b])

    def wait_gather(b):
        pltpu.make_async_copy(h_hbm.at[srcs.at[b]], rows.at[b], sems[b]).wait()

    def scatter(j, b):
        pltpu.sync_copy(rows.at[b], acc.at[dsts_pre.at[j]], add=True)

    load_src(0, 0)
    start_gather(0)

    def pair(jj, carry):
        j0 = 2 * jj
        load_src(j0 + 1, 1)
        start_gather(1)
        wait_gather(0)
        scatter(j0, 0)

        @pl.when(jj + 1 < CHUNKS_MSG // 2)
        def _():
            load_src(j0 + 2, 0)
            start_gather(0)

        wait_gather(1)
        scatter(j0 + 1, 1)
        return carry

    lax.fori_loop(0, CHUNKS_MSG // 2, pair, 0)
    plsc.subcore_barrier()
    pltpu.sync_copy(acc.at[pl.ds(sid * ROWS_PER_SUB, ROWS_PER_SUB)],
                    out_hbm.at[cid, pl.ds(sid * ROWS_PER_SUB, ROWS_PER_SUB)])


def _make_sc_msgpass(srow, drow):
    return pl.kernel(
        functools.partial(_sc_msgpass_body, srow=srow, drow=drow),
        out_type=jax.ShapeDtypeStruct((NC, N_PAD, D), jnp.float32),
        mesh=_mesh,
        scratch_types=[
            pltpu.VMEM((2, C), jnp.int32),
            pltpu.VMEM((CHUNKS_MSG, C), jnp.int32),
            pltpu.VMEM((2, C, D), jnp.float32),
            pltpu.VMEM_SHARED((N_PAD, D), jnp.float32),
            pltpu.SemaphoreType.DMA,
            pltpu.SemaphoreType.DMA,
        ],
    )


_sc_msgpass = _make_sc_msgpass(0, 1)

_BLK = 1024
_GRID = N_PAD // _BLK


def _tc_prescale_body(x_ref, deg_ref, o_ref):
    o_ref[...] = x_ref[...] * lax.rsqrt(jnp.maximum(deg_ref[0], 1.0))


_tc_prescale = pl.pallas_call(
    _tc_prescale_body,
    grid=(_GRID,),
    in_specs=[
        pl.BlockSpec((_BLK, D), lambda i: (i, 0)),
        pl.BlockSpec((NC, _BLK, D), lambda i: (0, i, 0)),
    ],
    out_specs=pl.BlockSpec((_BLK, D), lambda i: (i, 0)),
    out_shape=jax.ShapeDtypeStruct((N_PAD, D), jnp.float32),
)


def _tc_layer_body(p_ref, deg_ref, w_ref, b_ref, o_ref, *, last):
    m = (p_ref[0] + p_ref[1]) * lax.rsqrt(jnp.maximum(deg_ref[1], 1.0))
    y = jnp.dot(m, w_ref[...], preferred_element_type=jnp.float32) + b_ref[...]
    if last:
        o_ref[...] = jnp.max(y, axis=1, keepdims=True)
    else:
        o_ref[...] = jnp.maximum(y, 0.0) * lax.rsqrt(
            jnp.maximum(deg_ref[0], 1.0))


def _make_tc_layer(last):
    return pl.pallas_call(
        functools.partial(_tc_layer_body, last=last),
        grid=(_GRID,),
        in_specs=[
            pl.BlockSpec((NC, _BLK, D), lambda i: (0, i, 0)),
            pl.BlockSpec((NC, _BLK, D), lambda i: (0, i, 0)),
            pl.BlockSpec((D, D), lambda i: (0, 0)),
            pl.BlockSpec((1, D), lambda i: (0, 0)),
        ],
        out_specs=pl.BlockSpec((_BLK, 1 if last else D), lambda i: (i, 0)),
        out_shape=jax.ShapeDtypeStruct((N_PAD, 1 if last else D), jnp.float32),
    )


_tc_layer_mid = _make_tc_layer(last=False)
_tc_layer_last = _make_tc_layer(last=True)


def kernel(x, edge_index, W1, b1, W2, b2, W3, b3, W4, b4):
    ei = jnp.concatenate(
        [edge_index, jnp.full((2, E_PAD - E), N, dtype=jnp.int32)],
        axis=1).reshape(2, E_PAD // C, C)
    x_pad = jnp.concatenate(
        [x, jnp.zeros((N_PAD - N, D), dtype=jnp.float32)], axis=0)
    zeros128 = jnp.zeros((N_PAD, D), dtype=jnp.float32)

    deg = _sc_degree(ei)
    h = _tc_prescale(x_pad, deg)
    for w, b, last in ((W1, b1, False), (W2, b2, False), (W3, b3, False),
                       (W4, b4, True)):
        p = _sc_msgpass(h, ei, zeros128)
        layer = _tc_layer_last if last else _tc_layer_mid
        h = layer(p, deg, w, b.reshape(1, D))
    return h[:N, 0]
